# baseline (speedup 1.0000x reference)
import functools

import jax
import jax.numpy as jnp
from jax import lax
from jax.experimental import pallas as pl
from jax.experimental.pallas import tpu as pltpu
from jax.experimental.pallas import tpu_sc as plsc

B = 16384
FACTORS = 8
MLP_DIM = 32

NUM_CORES = 2
NUM_SUBCORES = 16
NW = NUM_CORES * NUM_SUBCORES
BPW = B // NW
CHUNK = 128
NCH = BPW // CHUNK


def _gather_body(user_hbm, item_hbm, mfu_t, mfi_t, mlu_t, mli_t,
                 mfu_o, mfi_o, mlu_o, mli_o,
                 idx_u, idx_i, mfu_v, mfi_v, mlu_v, mli_v, sem):
    wid = lax.axis_index("s") * NUM_CORES + lax.axis_index("c")
    base = wid * BPW
    row0 = wid * NCH
    pltpu.sync_copy(user_hbm.at[pl.ds(row0, NCH)], idx_u)
    pltpu.sync_copy(item_hbm.at[pl.ds(row0, NCH)], idx_i)
    copies = []
    for j in range(NCH):
        sl = pl.ds(j * CHUNK, CHUNK)
        copies.append(pltpu.async_copy(mfu_t.at[idx_u.at[j]], mfu_v.at[sl], sem))
        copies.append(pltpu.async_copy(mfi_t.at[idx_i.at[j]], mfi_v.at[sl], sem))
        copies.append(pltpu.async_copy(mlu_t.at[idx_u.at[j]], mlu_v.at[sl], sem))
        copies.append(pltpu.async_copy(mli_t.at[idx_i.at[j]], mli_v.at[sl], sem))
    for c in copies:
        c.wait()
    out_sl = pl.ds(base, BPW)
    pltpu.sync_copy(mfu_v, mfu_o.at[out_sl])
    pltpu.sync_copy(mfi_v, mfi_o.at[out_sl])
    pltpu.sync_copy(mlu_v, mlu_o.at[out_sl])
    pltpu.sync_copy(mli_v, mli_o.at[out_sl])


@jax.jit
def _sc_gather(user2d, item2d, mfu_t, mfi_t, mlu_t, mli_t):
    mesh = plsc.VectorSubcoreMesh(core_axis_name="c", subcore_axis_name="s")
    f32 = jnp.float32
    k = pl.kernel(
        _gather_body,
        mesh=mesh,
        out_type=(
            jax.ShapeDtypeStruct((B, FACTORS), f32),
            jax.ShapeDtypeStruct((B, FACTORS), f32),
            jax.ShapeDtypeStruct((B, MLP_DIM), f32),
            jax.ShapeDtypeStruct((B, MLP_DIM), f32),
        ),
        scratch_types=[
            pltpu.VMEM((NCH, CHUNK), jnp.int32),
            pltpu.VMEM((NCH, CHUNK), jnp.int32),
            pltpu.VMEM((BPW, FACTORS), f32),
            pltpu.VMEM((BPW, FACTORS), f32),
            pltpu.VMEM((BPW, MLP_DIM), f32),
            pltpu.VMEM((BPW, MLP_DIM), f32),
            pltpu.SemaphoreType.DMA,
        ],
        compiler_params=pltpu.CompilerParams(use_tc_tiling_on_sc=False),
    )
    return k(user2d, item2d, mfu_t, mfi_t, mlu_t, mli_t)


def _mlp_body(mfu, mfi, mlu, mli, ts, m0, m1,
              time_W, time_b, meta_W0, meta_b0, meta_W1, meta_b1,
              W0, b0, W1, b1, W2, b2, W3, b3, pW_mf, pW_mlp, pb, out):
    te = ts[...] * time_W[...] + time_b[...]
    me0 = m0[...] * meta_W0[...] + meta_b0[...]
    me1 = m1[...] * meta_W1[...] + meta_b1[...]
    x = jnp.concatenate([mlu[...], mli[...], te, me0, me1], axis=1)
    x = jnp.maximum(jnp.dot(x, W0[...], preferred_element_type=jnp.float32) + b0[...], 0.0)
    x = jnp.maximum(jnp.dot(x, W1[...], preferred_element_type=jnp.float32) + b1[...], 0.0)
    x = jnp.maximum(jnp.dot(x, W2[...], preferred_element_type=jnp.float32) + b2[...], 0.0)
    x = jnp.maximum(jnp.dot(x, W3[...], preferred_element_type=jnp.float32) + b3[...], 0.0)
    mfv = mfu[...] * mfi[...]
    logits = (jnp.dot(mfv, pW_mf[...], preferred_element_type=jnp.float32)
              + jnp.dot(x, pW_mlp[...], preferred_element_type=jnp.float32)
              + pb[...])
    out[...] = jax.nn.sigmoid(logits)


def kernel(user, item, timestamp, metadata, mf_user_emb, mf_item_emb,
           mlp_user_emb, mlp_item_emb, time_W, time_b, meta_Ws, meta_bs,
           mlp_Ws, mlp_bs, pred_W, pred_b):
    user2d = user.astype(jnp.int32).reshape(NW * NCH, CHUNK)
    item2d = item.astype(jnp.int32).reshape(NW * NCH, CHUNK)
    mfu, mfi, mlu, mli = _sc_gather(user2d, item2d, mf_user_emb, mf_item_emb,
                                    mlp_user_emb, mlp_item_emb)
    ts = timestamp.astype(jnp.float32).reshape(B, 1)
    m0 = metadata[0].astype(jnp.float32).reshape(B, 1)
    m1 = metadata[1].astype(jnp.float32).reshape(B, 1)
    args = (mfu, mfi, mlu, mli, ts, m0, m1,
            time_W, time_b.reshape(1, -1),
            meta_Ws[0], meta_bs[0].reshape(1, -1),
            meta_Ws[1], meta_bs[1].reshape(1, -1),
            mlp_Ws[0], mlp_bs[0].reshape(1, -1),
            mlp_Ws[1], mlp_bs[1].reshape(1, -1),
            mlp_Ws[2], mlp_bs[2].reshape(1, -1),
            mlp_Ws[3], mlp_bs[3].reshape(1, -1),
            pred_W[:FACTORS], pred_W[FACTORS:], pred_b.reshape(1, 1))
    R = 2048
    def row_spec(d):
        return pl.BlockSpec((R, d), lambda i: (i, 0))
    def w_spec(shape):
        return pl.BlockSpec(shape, lambda i: (0,) * len(shape))
    in_specs = [row_spec(FACTORS), row_spec(FACTORS), row_spec(MLP_DIM),
                row_spec(MLP_DIM), row_spec(1), row_spec(1), row_spec(1)]
    in_specs += [w_spec(a.shape) for a in args[7:]]
    out = pl.pallas_call(
        _mlp_body,
        grid=(B // R,),
        in_specs=in_specs,
        out_specs=pl.BlockSpec((R, 1), lambda i: (i, 0)),
        out_shape=jax.ShapeDtypeStruct((B, 1), jnp.float32),
    )(*args)
    return out.reshape(-1)


# speedup vs baseline: 1.2096x; 1.2096x over previous
import jax
import jax.numpy as jnp
from jax import lax
from jax.experimental import pallas as pl
from jax.experimental.pallas import tpu as pltpu

B = 16384
FACTORS = 8
MLP_DIM = 32
BLK = 2048
GRID = B // BLK


def _ncf_body(user_s, item_s, mlu_t, mli_t, mfu_t, mfi_t, ts, m0, m1,
              time_W, time_b, meta_W0, meta_b0, meta_W1, meta_b1,
              W0, b0, W1, b1, W2, b2, W3, b3, pW_mf, pW_mlp, pb,
              out, mlu_v, mli_v, mfu_v, mfi_v, sem):
    base = pl.program_id(0) * BLK

    def fire(r, c):
        u = user_s[base + r]
        it = item_s[base + r]
        pltpu.make_async_copy(mlu_t.at[pl.ds(u, 1)], mlu_v.at[pl.ds(r, 1)], sem).start()
        pltpu.make_async_copy(mli_t.at[pl.ds(it, 1)], mli_v.at[pl.ds(r, 1)], sem).start()
        pltpu.make_async_copy(mfu_t.at[pl.ds(u, 1)], mfu_v.at[pl.ds(r, 1)], sem).start()
        pltpu.make_async_copy(mfi_t.at[pl.ds(it, 1)], mfi_v.at[pl.ds(r, 1)], sem).start()
        return c

    lax.fori_loop(0, BLK, fire, 0, unroll=8)
    pltpu.make_async_copy(mlu_t.at[pl.ds(0, BLK)], mlu_v, sem).wait()
    pltpu.make_async_copy(mli_t.at[pl.ds(0, BLK)], mli_v, sem).wait()
    pltpu.make_async_copy(mfu_t.at[pl.ds(0, BLK)], mfu_v, sem).wait()
    pltpu.make_async_copy(mfi_t.at[pl.ds(0, BLK)], mfi_v, sem).wait()

    te = ts[...] * time_W[...] + time_b[...]
    me0 = m0[...] * meta_W0[...] + meta_b0[...]
    me1 = m1[...] * meta_W1[...] + meta_b1[...]
    x = jnp.concatenate([mlu_v[...], mli_v[...], te, me0, me1], axis=1)
    x = jnp.maximum(jnp.dot(x, W0[...], preferred_element_type=jnp.float32) + b0[...], 0.0)
    x = jnp.maximum(jnp.dot(x, W1[...], preferred_element_type=jnp.float32) + b1[...], 0.0)
    x = jnp.maximum(jnp.dot(x, W2[...], preferred_element_type=jnp.float32) + b2[...], 0.0)
    x = jnp.maximum(jnp.dot(x, W3[...], preferred_element_type=jnp.float32) + b3[...], 0.0)
    mfv = mfu_v[...] * mfi_v[...]
    logits = (jnp.dot(mfv, pW_mf[...], preferred_element_type=jnp.float32)
              + jnp.dot(x, pW_mlp[...], preferred_element_type=jnp.float32)
              + pb[...])
    out[...] = jax.nn.sigmoid(logits)


def kernel(user, item, timestamp, metadata, mf_user_emb, mf_item_emb,
           mlp_user_emb, mlp_item_emb, time_W, time_b, meta_Ws, meta_bs,
           mlp_Ws, mlp_bs, pred_W, pred_b):
    ts = timestamp.astype(jnp.float32).reshape(B, 1)
    m0 = metadata[0].astype(jnp.float32).reshape(B, 1)
    m1 = metadata[1].astype(jnp.float32).reshape(B, 1)
    weights = (time_W, time_b.reshape(1, -1),
               meta_Ws[0], meta_bs[0].reshape(1, -1),
               meta_Ws[1], meta_bs[1].reshape(1, -1),
               mlp_Ws[0], mlp_bs[0].reshape(1, -1),
               mlp_Ws[1], mlp_bs[1].reshape(1, -1),
               mlp_Ws[2], mlp_bs[2].reshape(1, -1),
               mlp_Ws[3], mlp_bs[3].reshape(1, -1),
               pred_W[:FACTORS], pred_W[FACTORS:], pred_b.reshape(1, 1))

    def hbm_spec():
        return pl.BlockSpec(memory_space=pltpu.MemorySpace.HBM)
    def row_spec(d):
        return pl.BlockSpec((BLK, d), lambda i, *_: (i, 0))
    def w_spec(shape):
        return pl.BlockSpec(shape, lambda i, *_: (0,) * len(shape))

    in_specs = [hbm_spec(), hbm_spec(), hbm_spec(), hbm_spec(),
                row_spec(1), row_spec(1), row_spec(1)]
    in_specs += [w_spec(a.shape) for a in weights]

    grid_spec = pltpu.PrefetchScalarGridSpec(
        num_scalar_prefetch=2,
        grid=(GRID,),
        in_specs=in_specs,
        out_specs=pl.BlockSpec((BLK, 1), lambda i, *_: (i, 0)),
        scratch_shapes=[
            pltpu.VMEM((BLK, MLP_DIM), jnp.float32),
            pltpu.VMEM((BLK, MLP_DIM), jnp.float32),
            pltpu.VMEM((BLK, FACTORS), jnp.float32),
            pltpu.VMEM((BLK, FACTORS), jnp.float32),
            pltpu.SemaphoreType.DMA,
        ],
    )
    out = pl.pallas_call(
        _ncf_body,
        grid_spec=grid_spec,
        out_shape=jax.ShapeDtypeStruct((B, 1), jnp.float32),
    )(user.astype(jnp.int32), item.astype(jnp.int32),
      mlp_user_emb, mlp_item_emb, mf_user_emb, mf_item_emb,
      ts, m0, m1, *weights)
    return out.reshape(-1)


# speedup vs baseline: 1.4718x; 1.2167x over previous
import jax
import jax.numpy as jnp
from jax import lax
from jax.experimental import pallas as pl
from jax.experimental.pallas import tpu as pltpu
from jax.experimental.pallas import tpu_sc as plsc

B = 16384
FACTORS = 8
MLP_DIM = 32

NUM_CORES = 2
NUM_SUBCORES = 16
NW = NUM_CORES * NUM_SUBCORES
BPW = B // NW
CH = 128


def _gather_body(user_hbm, item_hbm, mlu_t, mli_t, mfu_t, mfi_t,
                 mlu_o, mli_o, mfu_o, mfi_o,
                 idx_u_v, idx_i_v,
                 mlu_v, mli_v, mfu_v, mfi_v, sem):
    wid = lax.axis_index("s") * NUM_CORES + lax.axis_index("c")
    base = wid * BPW
    pltpu.sync_copy(user_hbm.at[pl.ds(base, BPW)], idx_u_v)
    pltpu.sync_copy(item_hbm.at[pl.ds(base, BPW)], idx_i_v)

    for ch in range(BPW // CH):
        def fire(c, carry):
            off = ch * CH + c * 16
            u_vec = idx_u_v[pl.ds(off, 16)]
            i_vec = idx_i_v[pl.ds(off, 16)]
            for j in range(16):
                u = u_vec[j]
                it = i_vec[j]
                r = c * 16 + j
                pltpu.async_copy(mlu_t.at[pl.ds(u, 1)], mlu_v.at[pl.ds(r, 1)], sem)
                pltpu.async_copy(mli_t.at[pl.ds(it, 1)], mli_v.at[pl.ds(r, 1)], sem)
                pltpu.async_copy(mfu_t.at[pl.ds(u, 1)], mfu_v.at[pl.ds(r, 1)], sem)
                pltpu.async_copy(mfi_t.at[pl.ds(it, 1)], mfi_v.at[pl.ds(r, 1)], sem)
            return carry

        lax.fori_loop(0, CH // 16, fire, 0, unroll=False)
        pltpu.make_async_copy(mlu_t.at[pl.ds(0, CH)], mlu_v, sem).wait()
        pltpu.make_async_copy(mli_t.at[pl.ds(0, CH)], mli_v, sem).wait()
        pltpu.make_async_copy(mfu_t.at[pl.ds(0, CH)], mfu_v, sem).wait()
        pltpu.make_async_copy(mfi_t.at[pl.ds(0, CH)], mfi_v, sem).wait()

        out_sl = pl.ds(base + ch * CH, CH)
        pltpu.sync_copy(mlu_v, mlu_o.at[out_sl])
        pltpu.sync_copy(mli_v, mli_o.at[out_sl])
        pltpu.sync_copy(mfu_v, mfu_o.at[out_sl])
        pltpu.sync_copy(mfi_v, mfi_o.at[out_sl])


def _sc_gather(user1d, item1d, mlu_t, mli_t, mfu_t, mfi_t):
    mesh = plsc.VectorSubcoreMesh(core_axis_name="c", subcore_axis_name="s")
    f32 = jnp.float32
    k = pl.kernel(
        _gather_body,
        mesh=mesh,
        out_type=(
            jax.ShapeDtypeStruct((B, MLP_DIM), f32),
            jax.ShapeDtypeStruct((B, MLP_DIM), f32),
            jax.ShapeDtypeStruct((B, FACTORS), f32),
            jax.ShapeDtypeStruct((B, FACTORS), f32),
        ),
        scratch_types=[
            pltpu.VMEM((BPW,), jnp.int32),
            pltpu.VMEM((BPW,), jnp.int32),
            pltpu.VMEM((CH, MLP_DIM), f32),
            pltpu.VMEM((CH, MLP_DIM), f32),
            pltpu.VMEM((CH, FACTORS), f32),
            pltpu.VMEM((CH, FACTORS), f32),
            pltpu.SemaphoreType.DMA,
        ],
    )
    return k(user1d, item1d, mlu_t, mli_t, mfu_t, mfi_t)


def _mlp_body(mlu, mli, mfu, mfi, ts, m0, m1,
              time_W, time_b, meta_W0, meta_b0, meta_W1, meta_b1,
              W0, b0, W1, b1, W2, b2, W3, b3, pW_mf, pW_mlp, pb, out):
    te = ts[...] * time_W[...] + time_b[...]
    me0 = m0[...] * meta_W0[...] + meta_b0[...]
    me1 = m1[...] * meta_W1[...] + meta_b1[...]
    x = jnp.concatenate([mlu[...], mli[...], te, me0, me1], axis=1)
    x = jnp.maximum(jnp.dot(x, W0[...], preferred_element_type=jnp.float32) + b0[...], 0.0)
    x = jnp.maximum(jnp.dot(x, W1[...], preferred_element_type=jnp.float32) + b1[...], 0.0)
    x = jnp.maximum(jnp.dot(x, W2[...], preferred_element_type=jnp.float32) + b2[...], 0.0)
    x = jnp.maximum(jnp.dot(x, W3[...], preferred_element_type=jnp.float32) + b3[...], 0.0)
    mfv = mfu[...] * mfi[...]
    logits = (jnp.dot(mfv, pW_mf[...], preferred_element_type=jnp.float32)
              + jnp.dot(x, pW_mlp[...], preferred_element_type=jnp.float32)
              + pb[...])
    out[...] = jax.nn.sigmoid(logits)


def kernel(user, item, timestamp, metadata, mf_user_emb, mf_item_emb,
           mlp_user_emb, mlp_item_emb, time_W, time_b, meta_Ws, meta_bs,
           mlp_Ws, mlp_bs, pred_W, pred_b):
    mlu, mli, mfu, mfi = _sc_gather(user.astype(jnp.int32), item.astype(jnp.int32),
                                    mlp_user_emb, mlp_item_emb,
                                    mf_user_emb, mf_item_emb)
    ts = timestamp.astype(jnp.float32).reshape(B, 1)
    m0 = metadata[0].astype(jnp.float32).reshape(B, 1)
    m1 = metadata[1].astype(jnp.float32).reshape(B, 1)
    args = (mlu, mli, mfu, mfi, ts, m0, m1,
            time_W, time_b.reshape(1, -1),
            meta_Ws[0], meta_bs[0].reshape(1, -1),
            meta_Ws[1], meta_bs[1].reshape(1, -1),
            mlp_Ws[0], mlp_bs[0].reshape(1, -1),
            mlp_Ws[1], mlp_bs[1].reshape(1, -1),
            mlp_Ws[2], mlp_bs[2].reshape(1, -1),
            mlp_Ws[3], mlp_bs[3].reshape(1, -1),
            pred_W[:FACTORS], pred_W[FACTORS:], pred_b.reshape(1, 1))
    R = 2048
    def row_spec(d):
        return pl.BlockSpec((R, d), lambda i: (i, 0))
    def w_spec(shape):
        return pl.BlockSpec(shape, lambda i: (0,) * len(shape))
    in_specs = [row_spec(MLP_DIM), row_spec(MLP_DIM), row_spec(FACTORS),
                row_spec(FACTORS), row_spec(1), row_spec(1), row_spec(1)]
    in_specs += [w_spec(a.shape) for a in args[7:]]
    out = pl.pallas_call(
        _mlp_body,
        grid=(B // R,),
        in_specs=in_specs,
        out_specs=pl.BlockSpec((R, 1), lambda i: (i, 0)),
        out_shape=jax.ShapeDtypeStruct((B, 1), jnp.float32),
    )(*args)
    return out.reshape(-1)


# speedup vs baseline: 1.4723x; 1.0003x over previous
import jax
import jax.numpy as jnp
from jax import lax
from jax.experimental import pallas as pl
from jax.experimental.pallas import tpu as pltpu
from jax.experimental.pallas import tpu_sc as plsc

B = 16384
FACTORS = 8
MLP_DIM = 32

NUM_CORES = 2
NUM_SUBCORES = 16
NW = NUM_CORES * NUM_SUBCORES
BPW = B // NW
CH = 128


def _gather_body(user_hbm, item_hbm, mlu_t, mli_t, mfu_t, mfi_t,
                 mlu_o, mli_o, mfu_o, mfi_o,
                 idx_u_v, idx_i_v,
                 mlu_v, mli_v, mfu_v, mfi_v, sem):
    wid = lax.axis_index("s") * NUM_CORES + lax.axis_index("c")
    base = wid * BPW
    pltpu.sync_copy(user_hbm.at[pl.ds(base, BPW)], idx_u_v)
    pltpu.sync_copy(item_hbm.at[pl.ds(base, BPW)], idx_i_v)

    for ch in range(BPW // CH):
        def fire(c, carry):
            off = ch * CH + c * 16
            u_vec = idx_u_v[pl.ds(off, 16)]
            i_vec = idx_i_v[pl.ds(off, 16)]
            for j in range(16):
                u = u_vec[j]
                it = i_vec[j]
                r = c * 16 + j
                pltpu.async_copy(mlu_t.at[pl.ds(u, 1)], mlu_v.at[pl.ds(r, 1)], sem)
                pltpu.async_copy(mli_t.at[pl.ds(it, 1)], mli_v.at[pl.ds(r, 1)], sem)
                pltpu.async_copy(mfu_t.at[pl.ds(u, 1)], mfu_v.at[pl.ds(r, 1)], sem)
                pltpu.async_copy(mfi_t.at[pl.ds(it, 1)], mfi_v.at[pl.ds(r, 1)], sem)
            return carry

        lax.fori_loop(0, CH // 16, fire, 0, unroll=False)
        pltpu.make_async_copy(mlu_t.at[pl.ds(0, CH)], mlu_v, sem).wait()
        pltpu.make_async_copy(mli_t.at[pl.ds(0, CH)], mli_v, sem).wait()
        pltpu.make_async_copy(mfu_t.at[pl.ds(0, CH)], mfu_v, sem).wait()
        pltpu.make_async_copy(mfi_t.at[pl.ds(0, CH)], mfi_v, sem).wait()

        out_sl = pl.ds(base + ch * CH, CH)
        pltpu.sync_copy(mlu_v, mlu_o.at[out_sl])
        pltpu.sync_copy(mli_v, mli_o.at[out_sl])
        pltpu.sync_copy(mfu_v, mfu_o.at[out_sl])
        pltpu.sync_copy(mfi_v, mfi_o.at[out_sl])


def _sc_gather(user1d, item1d, mlu_t, mli_t, mfu_t, mfi_t):
    mesh = plsc.VectorSubcoreMesh(core_axis_name="c", subcore_axis_name="s")
    f32 = jnp.float32
    k = pl.kernel(
        _gather_body,
        mesh=mesh,
        out_type=(
            jax.ShapeDtypeStruct((B, MLP_DIM), f32),
            jax.ShapeDtypeStruct((B, MLP_DIM), f32),
            jax.ShapeDtypeStruct((B, FACTORS), f32),
            jax.ShapeDtypeStruct((B, FACTORS), f32),
        ),
        scratch_types=[
            pltpu.VMEM((BPW,), jnp.int32),
            pltpu.VMEM((BPW,), jnp.int32),
            pltpu.VMEM((CH, MLP_DIM), f32),
            pltpu.VMEM((CH, MLP_DIM), f32),
            pltpu.VMEM((CH, FACTORS), f32),
            pltpu.VMEM((CH, FACTORS), f32),
            pltpu.SemaphoreType.DMA,
        ],
        compiler_params=pltpu.CompilerParams(skip_device_barrier=True),
    )
    return k(user1d, item1d, mlu_t, mli_t, mfu_t, mfi_t)


def _mlp_body(mlu, mli, mfu, mfi, ts, m0, m1,
              time_W, time_b, meta_W0, meta_b0, meta_W1, meta_b1,
              W0, b0, W1, b1, W2, b2, W3, b3, pW_mf, pW_mlp, pb, out):
    te = ts[...] * time_W[...] + time_b[...]
    me0 = m0[...] * meta_W0[...] + meta_b0[...]
    me1 = m1[...] * meta_W1[...] + meta_b1[...]
    x = jnp.concatenate([mlu[...], mli[...], te, me0, me1], axis=1)
    x = jnp.maximum(jnp.dot(x, W0[...], preferred_element_type=jnp.float32) + b0[...], 0.0)
    x = jnp.maximum(jnp.dot(x, W1[...], preferred_element_type=jnp.float32) + b1[...], 0.0)
    x = jnp.maximum(jnp.dot(x, W2[...], preferred_element_type=jnp.float32) + b2[...], 0.0)
    x = jnp.maximum(jnp.dot(x, W3[...], preferred_element_type=jnp.float32) + b3[...], 0.0)
    mfv = mfu[...] * mfi[...]
    logits = (jnp.dot(mfv, pW_mf[...], preferred_element_type=jnp.float32)
              + jnp.dot(x, pW_mlp[...], preferred_element_type=jnp.float32)
              + pb[...])
    out[...] = jax.nn.sigmoid(logits)


def kernel(user, item, timestamp, metadata, mf_user_emb, mf_item_emb,
           mlp_user_emb, mlp_item_emb, time_W, time_b, meta_Ws, meta_bs,
           mlp_Ws, mlp_bs, pred_W, pred_b):
    mlu, mli, mfu, mfi = _sc_gather(user.astype(jnp.int32), item.astype(jnp.int32),
                                    mlp_user_emb, mlp_item_emb,
                                    mf_user_emb, mf_item_emb)
    ts = timestamp.astype(jnp.float32).reshape(B, 1)
    m0 = metadata[0].astype(jnp.float32).reshape(B, 1)
    m1 = metadata[1].astype(jnp.float32).reshape(B, 1)
    args = (mlu, mli, mfu, mfi, ts, m0, m1,
            time_W, time_b.reshape(1, -1),
            meta_Ws[0], meta_bs[0].reshape(1, -1),
            meta_Ws[1], meta_bs[1].reshape(1, -1),
            mlp_Ws[0], mlp_bs[0].reshape(1, -1),
            mlp_Ws[1], mlp_bs[1].reshape(1, -1),
            mlp_Ws[2], mlp_bs[2].reshape(1, -1),
            mlp_Ws[3], mlp_bs[3].reshape(1, -1),
            pred_W[:FACTORS], pred_W[FACTORS:], pred_b.reshape(1, 1))
    R = 2048
    def row_spec(d):
        return pl.BlockSpec((R, d), lambda i: (i, 0))
    def w_spec(shape):
        return pl.BlockSpec(shape, lambda i: (0,) * len(shape))
    in_specs = [row_spec(MLP_DIM), row_spec(MLP_DIM), row_spec(FACTORS),
                row_spec(FACTORS), row_spec(1), row_spec(1), row_spec(1)]
    in_specs += [w_spec(a.shape) for a in args[7:]]
    out = pl.pallas_call(
        _mlp_body,
        grid=(B // R,),
        in_specs=in_specs,
        out_specs=pl.BlockSpec((R, 1), lambda i: (i, 0)),
        out_shape=jax.ShapeDtypeStruct((B, 1), jnp.float32),
    )(*args)
    return out.reshape(-1)


# speedup vs baseline: 1.4807x; 1.0058x over previous
import jax
import jax.numpy as jnp
from jax import lax
from jax.experimental import pallas as pl
from jax.experimental.pallas import tpu as pltpu
from jax.experimental.pallas import tpu_sc as plsc

B = 16384
FACTORS = 8
MLP_DIM = 32

NUM_CORES = 2
NUM_SUBCORES = 16
NW = NUM_CORES * NUM_SUBCORES
BPW = B // NW
CH = 128


def _gather_body(user_hbm, item_hbm, mlu_t, mli_t, mfu_t, mfi_t,
                 mlu_o, mli_o, mfu_o, mfi_o,
                 idx_u_v, idx_i_v,
                 mlu_v, mli_v, mfu_v, mfi_v, sem):
    wid = lax.axis_index("s") * NUM_CORES + lax.axis_index("c")
    base = wid * BPW
    pltpu.sync_copy(user_hbm.at[pl.ds(base, BPW)], idx_u_v)
    pltpu.sync_copy(item_hbm.at[pl.ds(base, BPW)], idx_i_v)

    for ch in range(BPW // CH):
        def fire(c, carry):
            off = ch * CH + c * 16
            u_vec = idx_u_v[pl.ds(off, 16)]
            i_vec = idx_i_v[pl.ds(off, 16)]
            for j in range(16):
                u = u_vec[j]
                it = i_vec[j]
                r = c * 16 + j
                pltpu.async_copy(mlu_t.at[pl.ds(u, 1)], mlu_v.at[pl.ds(r, 1)], sem)
                pltpu.async_copy(mli_t.at[pl.ds(it, 1)], mli_v.at[pl.ds(r, 1)], sem)
                pltpu.async_copy(mfu_t.at[pl.ds(u, 1)], mfu_v.at[pl.ds(r, 1)], sem)
                pltpu.async_copy(mfi_t.at[pl.ds(it, 1)], mfi_v.at[pl.ds(r, 1)], sem)
            return carry

        lax.fori_loop(0, CH // 16, fire, 0, unroll=False)
        pltpu.make_async_copy(mlu_t.at[pl.ds(0, CH)], mlu_v, sem).wait()
        pltpu.make_async_copy(mli_t.at[pl.ds(0, CH)], mli_v, sem).wait()
        pltpu.make_async_copy(mfu_t.at[pl.ds(0, CH)], mfu_v, sem).wait()
        pltpu.make_async_copy(mfi_t.at[pl.ds(0, CH)], mfi_v, sem).wait()

        out_sl = pl.ds(base + ch * CH, CH)
        pltpu.sync_copy(mlu_v, mlu_o.at[out_sl])
        pltpu.sync_copy(mli_v, mli_o.at[out_sl])
        pltpu.sync_copy(mfu_v, mfu_o.at[out_sl])
        pltpu.sync_copy(mfi_v, mfi_o.at[out_sl])


def _sc_gather(user1d, item1d, mlu_t, mli_t, mfu_t, mfi_t):
    mesh = plsc.VectorSubcoreMesh(core_axis_name="c", subcore_axis_name="s")
    f32 = jnp.float32
    k = pl.kernel(
        _gather_body,
        mesh=mesh,
        out_type=(
            jax.ShapeDtypeStruct((B, MLP_DIM), f32),
            jax.ShapeDtypeStruct((B, MLP_DIM), f32),
            jax.ShapeDtypeStruct((B, FACTORS), f32),
            jax.ShapeDtypeStruct((B, FACTORS), f32),
        ),
        scratch_types=[
            pltpu.VMEM((BPW,), jnp.int32),
            pltpu.VMEM((BPW,), jnp.int32),
            pltpu.VMEM((CH, MLP_DIM), f32),
            pltpu.VMEM((CH, MLP_DIM), f32),
            pltpu.VMEM((CH, FACTORS), f32),
            pltpu.VMEM((CH, FACTORS), f32),
            pltpu.SemaphoreType.DMA,
        ],
        compiler_params=pltpu.CompilerParams(skip_device_barrier=True),
    )
    return k(user1d, item1d, mlu_t, mli_t, mfu_t, mfi_t)


R = 4096


def _mlp_body(mlu, mli, mfu, mfi, ts, meta,
              time_W, time_b, meta_W0, meta_b0, meta_W1, meta_b1,
              W0, b0, W1, b1, W2, b2, W3, b3, pW, pb, out):
    tsc = ts[...].reshape(R, 1)
    m = meta[...]
    m0c = m[0, :].reshape(R, 1)
    m1c = m[1, :].reshape(R, 1)
    te = tsc * time_W[...] + time_b[...][None, :]
    me0 = m0c * meta_W0[...] + meta_b0[...][None, :]
    me1 = m1c * meta_W1[...] + meta_b1[...][None, :]
    x = jnp.concatenate([mlu[...], mli[...], te, me0, me1], axis=1)
    x = jnp.maximum(jnp.dot(x, W0[...], preferred_element_type=jnp.float32) + b0[...][None, :], 0.0)
    x = jnp.maximum(jnp.dot(x, W1[...], preferred_element_type=jnp.float32) + b1[...][None, :], 0.0)
    x = jnp.maximum(jnp.dot(x, W2[...], preferred_element_type=jnp.float32) + b2[...][None, :], 0.0)
    x = jnp.maximum(jnp.dot(x, W3[...], preferred_element_type=jnp.float32) + b3[...][None, :], 0.0)
    mfv = mfu[...] * mfi[...]
    pw = pW[...]
    logits = (jnp.dot(mfv, pw[:FACTORS], preferred_element_type=jnp.float32)
              + jnp.dot(x, pw[FACTORS:], preferred_element_type=jnp.float32)
              + pb[...][None, :])
    out[...] = jax.nn.sigmoid(logits).reshape(R)


def kernel(user, item, timestamp, metadata, mf_user_emb, mf_item_emb,
           mlp_user_emb, mlp_item_emb, time_W, time_b, meta_Ws, meta_bs,
           mlp_Ws, mlp_bs, pred_W, pred_b):
    mlu, mli, mfu, mfi = _sc_gather(user.astype(jnp.int32), item.astype(jnp.int32),
                                    mlp_user_emb, mlp_item_emb,
                                    mf_user_emb, mf_item_emb)
    args = (mlu, mli, mfu, mfi,
            timestamp, metadata,
            time_W, time_b, meta_Ws[0], meta_bs[0], meta_Ws[1], meta_bs[1],
            mlp_Ws[0], mlp_bs[0], mlp_Ws[1], mlp_bs[1],
            mlp_Ws[2], mlp_bs[2], mlp_Ws[3], mlp_bs[3],
            pred_W, pred_b)
    def row_spec(d):
        return pl.BlockSpec((R, d), lambda i: (i, 0))
    def w_spec(shape):
        return pl.BlockSpec(shape, lambda i: (0,) * len(shape))
    in_specs = [row_spec(MLP_DIM), row_spec(MLP_DIM), row_spec(FACTORS),
                row_spec(FACTORS),
                pl.BlockSpec((R,), lambda i: (i,)),
                pl.BlockSpec((2, R), lambda i: (0, i))]
    in_specs += [w_spec(a.shape) for a in args[6:]]
    out = pl.pallas_call(
        _mlp_body,
        grid=(B // R,),
        in_specs=in_specs,
        out_specs=pl.BlockSpec((R,), lambda i: (i,)),
        out_shape=jax.ShapeDtypeStruct((B,), jnp.float32),
    )(*args)
    return out


# speedup vs baseline: 1.4826x; 1.0013x over previous
import jax
import jax.numpy as jnp
from jax import lax
from jax.experimental import pallas as pl
from jax.experimental.pallas import tpu as pltpu
from jax.experimental.pallas import tpu_sc as plsc

B = 16384
FACTORS = 8
MLP_DIM = 32

NUM_CORES = 2
NUM_SUBCORES = 16
NW = NUM_CORES * NUM_SUBCORES
BPW = B // NW
CH = 128


def _gather_body(user_hbm, item_hbm, mlu_t, mli_t, mfu_t, mfi_t,
                 mlu_o, mli_o, mfu_o, mfi_o,
                 idx_u_v, idx_i_v,
                 mlu_v, mli_v, mfu_v, mfi_v, sem0, sem1, sem2, sem3):
    wid = lax.axis_index("s") * NUM_CORES + lax.axis_index("c")
    base = wid * BPW
    pltpu.sync_copy(user_hbm.at[pl.ds(base, BPW)], idx_u_v)
    pltpu.sync_copy(item_hbm.at[pl.ds(base, BPW)], idx_i_v)

    for ch in range(BPW // CH):
        def fire(c, carry):
            off = ch * CH + c * 16
            u_vec = idx_u_v[pl.ds(off, 16)]
            i_vec = idx_i_v[pl.ds(off, 16)]
            for j in range(16):
                u = u_vec[j]
                it = i_vec[j]
                r = c * 16 + j
                pltpu.async_copy(mlu_t.at[pl.ds(u, 1)], mlu_v.at[pl.ds(r, 1)], sem0)
                pltpu.async_copy(mli_t.at[pl.ds(it, 1)], mli_v.at[pl.ds(r, 1)], sem1)
                pltpu.async_copy(mfu_t.at[pl.ds(u, 1)], mfu_v.at[pl.ds(r, 1)], sem2)
                pltpu.async_copy(mfi_t.at[pl.ds(it, 1)], mfi_v.at[pl.ds(r, 1)], sem3)
            return carry

        lax.fori_loop(0, CH // 16, fire, 0, unroll=False)
        pltpu.make_async_copy(mlu_t.at[pl.ds(0, CH)], mlu_v, sem0).wait()
        pltpu.make_async_copy(mli_t.at[pl.ds(0, CH)], mli_v, sem1).wait()
        pltpu.make_async_copy(mfu_t.at[pl.ds(0, CH)], mfu_v, sem2).wait()
        pltpu.make_async_copy(mfi_t.at[pl.ds(0, CH)], mfi_v, sem3).wait()

        out_sl = pl.ds(base + ch * CH, CH)
        pltpu.sync_copy(mlu_v, mlu_o.at[out_sl])
        pltpu.sync_copy(mli_v, mli_o.at[out_sl])
        pltpu.sync_copy(mfu_v, mfu_o.at[out_sl])
        pltpu.sync_copy(mfi_v, mfi_o.at[out_sl])


def _sc_gather(user1d, item1d, mlu_t, mli_t, mfu_t, mfi_t):
    mesh = plsc.VectorSubcoreMesh(core_axis_name="c", subcore_axis_name="s")
    f32 = jnp.float32
    k = pl.kernel(
        _gather_body,
        mesh=mesh,
        out_type=(
            jax.ShapeDtypeStruct((B, MLP_DIM), f32),
            jax.ShapeDtypeStruct((B, MLP_DIM), f32),
            jax.ShapeDtypeStruct((B, FACTORS), f32),
            jax.ShapeDtypeStruct((B, FACTORS), f32),
        ),
        scratch_types=[
            pltpu.VMEM((BPW,), jnp.int32),
            pltpu.VMEM((BPW,), jnp.int32),
            pltpu.VMEM((CH, MLP_DIM), f32),
            pltpu.VMEM((CH, MLP_DIM), f32),
            pltpu.VMEM((CH, FACTORS), f32),
            pltpu.VMEM((CH, FACTORS), f32),
            pltpu.SemaphoreType.DMA,
            pltpu.SemaphoreType.DMA,
            pltpu.SemaphoreType.DMA,
            pltpu.SemaphoreType.DMA,
        ],
        compiler_params=pltpu.CompilerParams(skip_device_barrier=True),
    )
    return k(user1d, item1d, mlu_t, mli_t, mfu_t, mfi_t)


R = 4096


def _mlp_body(mlu, mli, mfu, mfi, ts, meta,
              time_W, time_b, meta_W0, meta_b0, meta_W1, meta_b1,
              W0, b0, W1, b1, W2, b2, W3, b3, pW, pb, out):
    tsc = ts[...].reshape(R, 1)
    m = meta[...]
    m0c = m[0, :].reshape(R, 1)
    m1c = m[1, :].reshape(R, 1)
    te = tsc * time_W[...] + time_b[...][None, :]
    me0 = m0c * meta_W0[...] + meta_b0[...][None, :]
    me1 = m1c * meta_W1[...] + meta_b1[...][None, :]
    x = jnp.concatenate([mlu[...], mli[...], te, me0, me1], axis=1)
    x = jnp.maximum(jnp.dot(x, W0[...], preferred_element_type=jnp.float32) + b0[...][None, :], 0.0)
    x = jnp.maximum(jnp.dot(x, W1[...], preferred_element_type=jnp.float32) + b1[...][None, :], 0.0)
    x = jnp.maximum(jnp.dot(x, W2[...], preferred_element_type=jnp.float32) + b2[...][None, :], 0.0)
    x = jnp.maximum(jnp.dot(x, W3[...], preferred_element_type=jnp.float32) + b3[...][None, :], 0.0)
    mfv = mfu[...] * mfi[...]
    pw = pW[...]
    logits = (jnp.dot(mfv, pw[:FACTORS], preferred_element_type=jnp.float32)
              + jnp.dot(x, pw[FACTORS:], preferred_element_type=jnp.float32)
              + pb[...][None, :])
    out[...] = jax.nn.sigmoid(logits).reshape(R)


def kernel(user, item, timestamp, metadata, mf_user_emb, mf_item_emb,
           mlp_user_emb, mlp_item_emb, time_W, time_b, meta_Ws, meta_bs,
           mlp_Ws, mlp_bs, pred_W, pred_b):
    mlu, mli, mfu, mfi = _sc_gather(user.astype(jnp.int32), item.astype(jnp.int32),
                                    mlp_user_emb, mlp_item_emb,
                                    mf_user_emb, mf_item_emb)
    args = (mlu, mli, mfu, mfi,
            timestamp, metadata,
            time_W, time_b, meta_Ws[0], meta_bs[0], meta_Ws[1], meta_bs[1],
            mlp_Ws[0], mlp_bs[0], mlp_Ws[1], mlp_bs[1],
            mlp_Ws[2], mlp_bs[2], mlp_Ws[3], mlp_bs[3],
            pred_W, pred_b)
    def row_spec(d):
        return pl.BlockSpec((R, d), lambda i: (i, 0))
    def w_spec(shape):
        return pl.BlockSpec(shape, lambda i: (0,) * len(shape))
    in_specs = [row_spec(MLP_DIM), row_spec(MLP_DIM), row_spec(FACTORS),
                row_spec(FACTORS),
                pl.BlockSpec((R,), lambda i: (i,)),
                pl.BlockSpec((2, R), lambda i: (0, i))]
    in_specs += [w_spec(a.shape) for a in args[6:]]
    out = pl.pallas_call(
        _mlp_body,
        grid=(B // R,),
        in_specs=in_specs,
        out_specs=pl.BlockSpec((R,), lambda i: (i,)),
        out_shape=jax.ShapeDtypeStruct((B,), jnp.float32),
    )(*args)
    return out


# speedup vs baseline: 1.4838x; 1.0008x over previous
import jax
import jax.numpy as jnp
from jax import lax
from jax.experimental import pallas as pl
from jax.experimental.pallas import tpu as pltpu
from jax.experimental.pallas import tpu_sc as plsc

B = 16384
FACTORS = 8
MLP_DIM = 32

NUM_CORES = 2
NUM_SUBCORES = 16
NW = NUM_CORES * NUM_SUBCORES
BPW = B // NW
CH = 64


def _gather_body(user_hbm, item_hbm, mlu_t, mli_t, mfu_t, mfi_t,
                 mlu_o, mli_o, mfu_o, mfi_o,
                 idx_u_v, idx_i_v,
                 mlu_v, mli_v, mfu_v, mfi_v, sem):
    wid = lax.axis_index("s") * NUM_CORES + lax.axis_index("c")
    base = wid * BPW
    pltpu.sync_copy(user_hbm.at[pl.ds(base, BPW)], idx_u_v)
    pltpu.sync_copy(item_hbm.at[pl.ds(base, BPW)], idx_i_v)

    def fire(ch, half):
        def body(c, carry):
            off = ch * CH + c * 16
            u_vec = idx_u_v[pl.ds(off, 16)]
            i_vec = idx_i_v[pl.ds(off, 16)]
            for j in range(16):
                u = u_vec[j]
                it = i_vec[j]
                r = half * CH + c * 16 + j
                pltpu.async_copy(mlu_t.at[pl.ds(u, 1)], mlu_v.at[pl.ds(r, 1)], sem)
                pltpu.async_copy(mli_t.at[pl.ds(it, 1)], mli_v.at[pl.ds(r, 1)], sem)
                pltpu.async_copy(mfu_t.at[pl.ds(u, 1)], mfu_v.at[pl.ds(r, 1)], sem)
                pltpu.async_copy(mfi_t.at[pl.ds(it, 1)], mfi_v.at[pl.ds(r, 1)], sem)
            return carry
        lax.fori_loop(0, CH // 16, body, 0, unroll=False)

    def drain_and_store(ch, half):
        sl = pl.ds(half * CH, CH)
        pltpu.make_async_copy(mlu_t.at[pl.ds(0, CH)], mlu_v.at[sl], sem).wait()
        pltpu.make_async_copy(mli_t.at[pl.ds(0, CH)], mli_v.at[sl], sem).wait()
        pltpu.make_async_copy(mfu_t.at[pl.ds(0, CH)], mfu_v.at[sl], sem).wait()
        pltpu.make_async_copy(mfi_t.at[pl.ds(0, CH)], mfi_v.at[sl], sem).wait()
        out_sl = pl.ds(base + ch * CH, CH)
        pltpu.sync_copy(mlu_v.at[sl], mlu_o.at[out_sl])
        pltpu.sync_copy(mli_v.at[sl], mli_o.at[out_sl])
        pltpu.sync_copy(mfu_v.at[sl], mfu_o.at[out_sl])
        pltpu.sync_copy(mfi_v.at[sl], mfi_o.at[out_sl])

    NCH = BPW // CH
    fire(0, 0)
    for ch in range(1, NCH):
        fire(ch, ch % 2)
        drain_and_store(ch - 1, (ch - 1) % 2)
    drain_and_store(NCH - 1, (NCH - 1) % 2)


def _sc_gather(user1d, item1d, mlu_t, mli_t, mfu_t, mfi_t):
    mesh = plsc.VectorSubcoreMesh(core_axis_name="c", subcore_axis_name="s")
    f32 = jnp.float32
    k = pl.kernel(
        _gather_body,
        mesh=mesh,
        out_type=(
            jax.ShapeDtypeStruct((B, MLP_DIM), f32),
            jax.ShapeDtypeStruct((B, MLP_DIM), f32),
            jax.ShapeDtypeStruct((B, FACTORS), f32),
            jax.ShapeDtypeStruct((B, FACTORS), f32),
        ),
        scratch_types=[
            pltpu.VMEM((BPW,), jnp.int32),
            pltpu.VMEM((BPW,), jnp.int32),
            pltpu.VMEM((2 * CH, MLP_DIM), f32),
            pltpu.VMEM((2 * CH, MLP_DIM), f32),
            pltpu.VMEM((2 * CH, FACTORS), f32),
            pltpu.VMEM((2 * CH, FACTORS), f32),
            pltpu.SemaphoreType.DMA,
        ],
        compiler_params=pltpu.CompilerParams(skip_device_barrier=True),
    )
    return k(user1d, item1d, mlu_t, mli_t, mfu_t, mfi_t)


R = 4096


def _mlp_body(mlu, mli, mfu, mfi, ts, meta,
              time_W, time_b, meta_W0, meta_b0, meta_W1, meta_b1,
              W0, b0, W1, b1, W2, b2, W3, b3, pW, pb, out):
    tsc = ts[...].reshape(R, 1)
    m = meta[...]
    m0c = m[0, :].reshape(R, 1)
    m1c = m[1, :].reshape(R, 1)
    te = tsc * time_W[...] + time_b[...][None, :]
    me0 = m0c * meta_W0[...] + meta_b0[...][None, :]
    me1 = m1c * meta_W1[...] + meta_b1[...][None, :]
    x = jnp.concatenate([mlu[...], mli[...], te, me0, me1], axis=1)
    x = jnp.maximum(jnp.dot(x, W0[...], preferred_element_type=jnp.float32) + b0[...][None, :], 0.0)
    x = jnp.maximum(jnp.dot(x, W1[...], preferred_element_type=jnp.float32) + b1[...][None, :], 0.0)
    x = jnp.maximum(jnp.dot(x, W2[...], preferred_element_type=jnp.float32) + b2[...][None, :], 0.0)
    x = jnp.maximum(jnp.dot(x, W3[...], preferred_element_type=jnp.float32) + b3[...][None, :], 0.0)
    mfv = mfu[...] * mfi[...]
    pw = pW[...]
    logits = (jnp.dot(mfv, pw[:FACTORS], preferred_element_type=jnp.float32)
              + jnp.dot(x, pw[FACTORS:], preferred_element_type=jnp.float32)
              + pb[...][None, :])
    out[...] = jax.nn.sigmoid(logits).reshape(R)


def kernel(user, item, timestamp, metadata, mf_user_emb, mf_item_emb,
           mlp_user_emb, mlp_item_emb, time_W, time_b, meta_Ws, meta_bs,
           mlp_Ws, mlp_bs, pred_W, pred_b):
    mlu, mli, mfu, mfi = _sc_gather(user.astype(jnp.int32), item.astype(jnp.int32),
                                    mlp_user_emb, mlp_item_emb,
                                    mf_user_emb, mf_item_emb)
    args = (mlu, mli, mfu, mfi,
            timestamp, metadata,
            time_W, time_b, meta_Ws[0], meta_bs[0], meta_Ws[1], meta_bs[1],
            mlp_Ws[0], mlp_bs[0], mlp_Ws[1], mlp_bs[1],
            mlp_Ws[2], mlp_bs[2], mlp_Ws[3], mlp_bs[3],
            pred_W, pred_b)
    def row_spec(d):
        return pl.BlockSpec((R, d), lambda i: (i, 0))
    def w_spec(shape):
        return pl.BlockSpec(shape, lambda i: (0,) * len(shape))
    in_specs = [row_spec(MLP_DIM), row_spec(MLP_DIM), row_spec(FACTORS),
                row_spec(FACTORS),
                pl.BlockSpec((R,), lambda i: (i,)),
                pl.BlockSpec((2, R), lambda i: (0, i))]
    in_specs += [w_spec(a.shape) for a in args[6:]]
    out = pl.pallas_call(
        _mlp_body,
        grid=(B // R,),
        in_specs=in_specs,
        out_specs=pl.BlockSpec((R,), lambda i: (i,)),
        out_shape=jax.ShapeDtypeStruct((B,), jnp.float32),
    )(*args)
    return out
